# Initial kernel scaffold; baseline (speedup 1.0000x reference)
import jax
import jax.numpy as jnp
from jax.experimental import pallas as pl


def kernel(text, text_lengths, emb, W1, b1, W2, b2):
    raise NotImplementedError("write your pallas kernel here")


# speedup vs baseline: 13.3342x; 13.3342x over previous
import functools

import jax
import jax.numpy as jnp
from jax import lax
from jax.experimental import pallas as pl
from jax.experimental.pallas import tpu as pltpu
from jax.experimental.pallas import tpu_sc as plsc

B = 16384
L = 200
HID = 32
NC = 2
NS = 16
NW = NC * NS
BPW = B // NW
NB = 8
GROUPS = BPW // NB
CHUNK = 100
CPB = L // CHUNK
NCHUNK = NB * CPB


def _pool_kernel(text_hbm, emb_hbm, out_hbm, idx_v, rows_v, out_v, gsem):
    wid = lax.axis_index("s") * NC + lax.axis_index("c")
    inv_l = jnp.float32(1.0 / L)

    @pl.loop(0, GROUPS)
    def _group(g):
        chunk_base = (wid * BPW + g * NB) * CPB
        pltpu.sync_copy(text_hbm.at[pl.ds(chunk_base, NCHUNK)], idx_v)
        copies = []
        for c in range(NCHUNK):
            copies.append(pltpu.async_copy(
                emb_hbm.at[idx_v.at[c]],
                rows_v.at[pl.ds(c * CHUNK, CHUNK)],
                gsem,
            ))
        for cp in copies:
            cp.wait()
        for i in range(NB):
            zero = jnp.zeros((16,), jnp.float32)

            @pl.loop(0, L, init_carry=(zero, zero), unroll=8)
            def _acc(j, carry):
                a0, a1 = carry
                r0 = rows_v[i * L + j, pl.ds(0, 16)]
                r1 = rows_v[i * L + j, pl.ds(16, 16)]
                return a0 + r0, a1 + r1

            a0, a1 = _acc
            slot = g * NB + i
            out_v[slot, pl.ds(0, 16)] = a0 * inv_l
            out_v[slot, pl.ds(16, 16)] = a1 * inv_l

    pltpu.sync_copy(out_v, out_hbm.at[pl.ds(wid * BPW, BPW)])


def _mlp_kernel(p_ref, w1_ref, b1_ref, w2_ref, b2_ref, o_ref):
    h = jnp.dot(p_ref[...], w1_ref[...],
                preferred_element_type=jnp.float32) + b1_ref[...]
    o_ref[...] = jnp.dot(h, w2_ref[...],
                         preferred_element_type=jnp.float32) + b2_ref[...]


def kernel(text, text_lengths, emb, W1, b1, W2, b2):
    del text_lengths
    text_chunks = text.astype(jnp.int32).reshape(B * L // CHUNK, CHUNK)

    mesh = plsc.VectorSubcoreMesh(core_axis_name="c", subcore_axis_name="s")
    pooled = pl.kernel(
        _pool_kernel,
        out_type=jax.ShapeDtypeStruct((B, HID), jnp.float32),
        mesh=mesh,
        compiler_params=pltpu.CompilerParams(use_tc_tiling_on_sc=False),
        scratch_types=[
            pltpu.VMEM((NCHUNK, CHUNK), jnp.int32),
            pltpu.VMEM((NB * L, HID), jnp.float32),
            pltpu.VMEM((BPW, HID), jnp.float32),
            pltpu.SemaphoreType.DMA,
        ],
    )(text_chunks, emb)

    BM = 2048
    NCLS = b2.shape[0]
    z = pl.pallas_call(
        _mlp_kernel,
        grid=(B // BM,),
        in_specs=[
            pl.BlockSpec((BM, HID), lambda i: (i, 0)),
            pl.BlockSpec((HID, HID), lambda i: (0, 0)),
            pl.BlockSpec((1, HID), lambda i: (0, 0)),
            pl.BlockSpec((HID, NCLS), lambda i: (0, 0)),
            pl.BlockSpec((1, NCLS), lambda i: (0, 0)),
        ],
        out_specs=pl.BlockSpec((BM, NCLS), lambda i: (i, 0)),
        out_shape=jax.ShapeDtypeStruct((B, NCLS), jnp.float32),
    )(pooled, W1, b1.reshape(1, HID), W2, b2.reshape(1, NCLS))
    return z


# speedup vs baseline: 16.0146x; 1.2010x over previous
import functools

import jax
import jax.numpy as jnp
from jax import lax
from jax.experimental import pallas as pl
from jax.experimental.pallas import tpu as pltpu
from jax.experimental.pallas import tpu_sc as plsc

B = 16384
L = 200
HID = 32
NC = 2
NS = 16
NW = NC * NS
BPW = B // NW
NB = 8
GROUPS = BPW // NB
CHUNK = 100
CPB = L // CHUNK
NCHUNK = NB * CPB
UNROLL = 8


def _pool_kernel(text_hbm, emb_hbm, out_hbm,
                 ibuf0, ibuf1, rbuf0, rbuf1, out_v,
                 isem0, isem1, gsem0, gsem1):
    wid = lax.axis_index("s") * NC + lax.axis_index("c")
    inv_l = jnp.float32(1.0 / L)
    zero = jnp.zeros((16,), jnp.float32)
    ibufs = (ibuf0, ibuf1)
    rbufs = (rbuf0, rbuf1)
    isems = (isem0, isem1)
    gsems = (gsem0, gsem1)

    def start_idx(g, b):
        chunk_base = (wid * BPW + g * NB) * CPB
        pltpu.async_copy(text_hbm.at[pl.ds(chunk_base, NCHUNK)],
                         ibufs[b], isems[b])

    def fire_gathers(b):
        pltpu.make_async_copy(text_hbm.at[pl.ds(0, NCHUNK)],
                              ibufs[b], isems[b]).wait()
        for c in range(NCHUNK):
            pltpu.async_copy(emb_hbm.at[ibufs[b].at[c]],
                             rbufs[b].at[pl.ds(c * CHUNK, CHUNK)], gsems[b])

    def drain_reduce(g, b):
        for c in range(NCHUNK):
            pltpu.make_async_copy(emb_hbm.at[ibufs[b].at[c]],
                                  rbufs[b].at[pl.ds(c * CHUNK, CHUNK)],
                                  gsems[b]).wait()
        rbuf = rbufs[b]
        for i in range(NB):
            @pl.loop(0, L, init_carry=(zero, zero), unroll=UNROLL)
            def _acc(j, carry):
                a0, a1 = carry
                r0 = rbuf[i * L + j, pl.ds(0, 16)]
                r1 = rbuf[i * L + j, pl.ds(16, 16)]
                return a0 + r0, a1 + r1

            a0, a1 = _acc
            slot = g * NB + i
            out_v[slot, pl.ds(0, 16)] = a0 * inv_l
            out_v[slot, pl.ds(16, 16)] = a1 * inv_l

    start_idx(0, 0)
    start_idx(1, 1)
    fire_gathers(0)

    @pl.loop(0, GROUPS, step=2)
    def _pair(g):
        fire_gathers(1)
        drain_reduce(g, 0)

        @pl.when(g + 2 < GROUPS)
        def _():
            start_idx(g + 2, 0)

        @pl.when(g + 2 < GROUPS)
        def _():
            fire_gathers(0)

        drain_reduce(g + 1, 1)

        @pl.when(g + 3 < GROUPS)
        def _():
            start_idx(g + 3, 1)

    pltpu.sync_copy(out_v, out_hbm.at[pl.ds(wid * BPW, BPW)])


def _mlp_kernel(p_ref, w1_ref, b1_ref, w2_ref, b2_ref, o_ref):
    h = jnp.dot(p_ref[...], w1_ref[...],
                preferred_element_type=jnp.float32) + b1_ref[...]
    o_ref[...] = jnp.dot(h, w2_ref[...],
                         preferred_element_type=jnp.float32) + b2_ref[...]


def kernel(text, text_lengths, emb, W1, b1, W2, b2):
    del text_lengths
    text_chunks = text.astype(jnp.int32).reshape(B * L // CHUNK, CHUNK)

    mesh = plsc.VectorSubcoreMesh(core_axis_name="c", subcore_axis_name="s")
    pooled = pl.kernel(
        _pool_kernel,
        out_type=jax.ShapeDtypeStruct((B, HID), jnp.float32),
        mesh=mesh,
        compiler_params=pltpu.CompilerParams(use_tc_tiling_on_sc=False),
        scratch_types=[
            pltpu.VMEM((NCHUNK, CHUNK), jnp.int32),
            pltpu.VMEM((NCHUNK, CHUNK), jnp.int32),
            pltpu.VMEM((NB * L, HID), jnp.float32),
            pltpu.VMEM((NB * L, HID), jnp.float32),
            pltpu.VMEM((BPW, HID), jnp.float32),
            pltpu.SemaphoreType.DMA,
            pltpu.SemaphoreType.DMA,
            pltpu.SemaphoreType.DMA,
            pltpu.SemaphoreType.DMA,
        ],
    )(text_chunks, emb)

    BM = 2048
    NCLS = b2.shape[0]
    z = pl.pallas_call(
        _mlp_kernel,
        grid=(B // BM,),
        in_specs=[
            pl.BlockSpec((BM, HID), lambda i: (i, 0)),
            pl.BlockSpec((HID, HID), lambda i: (0, 0)),
            pl.BlockSpec((1, HID), lambda i: (0, 0)),
            pl.BlockSpec((HID, NCLS), lambda i: (0, 0)),
            pl.BlockSpec((1, NCLS), lambda i: (0, 0)),
        ],
        out_specs=pl.BlockSpec((BM, NCLS), lambda i: (i, 0)),
        out_shape=jax.ShapeDtypeStruct((B, NCLS), jnp.float32),
    )(pooled, W1, b1.reshape(1, HID), W2, b2.reshape(1, NCLS))
    return z


# speedup vs baseline: 16.2447x; 1.0144x over previous
import functools

import jax
import jax.numpy as jnp
from jax import lax
from jax.experimental import pallas as pl
from jax.experimental.pallas import tpu as pltpu
from jax.experimental.pallas import tpu_sc as plsc

B = 16384
L = 200
HID = 32
NC = 2
NS = 16
NW = NC * NS
BPW = B // NW
NB = 8
GROUPS = BPW // NB
CHUNK = 64
NCHUNK = NB * L // CHUNK
UNROLL = 8


def _pool_kernel(text_hbm, emb_hbm, out_hbm,
                 ibuf0, ibuf1, rbuf0, rbuf1, out_v,
                 isem0, isem1, gsem0, gsem1):
    wid = lax.axis_index("s") * NC + lax.axis_index("c")
    inv_l = jnp.float32(1.0 / L)
    zero = jnp.zeros((16,), jnp.float32)
    ibufs = (ibuf0, ibuf1)
    rbufs = (rbuf0, rbuf1)
    isems = (isem0, isem1)
    gsems = (gsem0, gsem1)

    def start_idx(g, b):
        base = (wid * BPW + g * NB) * L
        pltpu.async_copy(text_hbm.at[pl.ds(base, NB * L)],
                         ibufs[b], isems[b])

    def fire_gathers(b):
        pltpu.make_async_copy(text_hbm.at[pl.ds(0, NB * L)],
                              ibufs[b], isems[b]).wait()
        for c in range(NCHUNK):
            pltpu.async_copy(emb_hbm.at[ibufs[b].at[pl.ds(c * CHUNK, CHUNK)]],
                             rbufs[b].at[pl.ds(c * CHUNK, CHUNK)], gsems[b])

    def drain_reduce(g, b):
        for c in range(NCHUNK):
            pltpu.make_async_copy(emb_hbm.at[ibufs[b].at[pl.ds(c * CHUNK, CHUNK)]],
                                  rbufs[b].at[pl.ds(c * CHUNK, CHUNK)],
                                  gsems[b]).wait()
        rbuf = rbufs[b]
        for i in range(NB):
            @pl.loop(0, L, init_carry=(zero, zero), unroll=UNROLL)
            def _acc(j, carry):
                a0, a1 = carry
                r0 = rbuf[i * L + j, pl.ds(0, 16)]
                r1 = rbuf[i * L + j, pl.ds(16, 16)]
                return a0 + r0, a1 + r1

            a0, a1 = _acc
            slot = g * NB + i
            out_v[slot, pl.ds(0, 16)] = a0 * inv_l
            out_v[slot, pl.ds(16, 16)] = a1 * inv_l

    start_idx(0, 0)
    start_idx(1, 1)
    fire_gathers(0)

    @pl.loop(0, GROUPS, step=2)
    def _pair(g):
        fire_gathers(1)
        drain_reduce(g, 0)

        @pl.when(g + 2 < GROUPS)
        def _():
            start_idx(g + 2, 0)

        @pl.when(g + 2 < GROUPS)
        def _():
            fire_gathers(0)

        drain_reduce(g + 1, 1)

        @pl.when(g + 3 < GROUPS)
        def _():
            start_idx(g + 3, 1)

    pltpu.sync_copy(out_v, out_hbm.at[pl.ds(wid * BPW, BPW)])


def _mlp_kernel(p_ref, w1_ref, b1_ref, w2_ref, b2_ref, o_ref):
    h = jnp.dot(p_ref[...], w1_ref[...],
                preferred_element_type=jnp.float32) + b1_ref[...]
    o_ref[...] = jnp.dot(h, w2_ref[...],
                         preferred_element_type=jnp.float32) + b2_ref[...]


def kernel(text, text_lengths, emb, W1, b1, W2, b2):
    del text_lengths
    text_flat = text.astype(jnp.int32).reshape(B * L)

    mesh = plsc.VectorSubcoreMesh(core_axis_name="c", subcore_axis_name="s")
    pooled = pl.kernel(
        _pool_kernel,
        out_type=jax.ShapeDtypeStruct((B, HID), jnp.float32),
        mesh=mesh,
        compiler_params=pltpu.CompilerParams(use_tc_tiling_on_sc=False),
        scratch_types=[
            pltpu.VMEM((NB * L,), jnp.int32),
            pltpu.VMEM((NB * L,), jnp.int32),
            pltpu.VMEM((NB * L, HID), jnp.float32),
            pltpu.VMEM((NB * L, HID), jnp.float32),
            pltpu.VMEM((BPW, HID), jnp.float32),
            pltpu.SemaphoreType.DMA,
            pltpu.SemaphoreType.DMA,
            pltpu.SemaphoreType.DMA,
            pltpu.SemaphoreType.DMA,
        ],
    )(text_flat, emb)

    BM = 2048
    NCLS = b2.shape[0]
    z = pl.pallas_call(
        _mlp_kernel,
        grid=(B // BM,),
        in_specs=[
            pl.BlockSpec((BM, HID), lambda i: (i, 0)),
            pl.BlockSpec((HID, HID), lambda i: (0, 0)),
            pl.BlockSpec((1, HID), lambda i: (0, 0)),
            pl.BlockSpec((HID, NCLS), lambda i: (0, 0)),
            pl.BlockSpec((1, NCLS), lambda i: (0, 0)),
        ],
        out_specs=pl.BlockSpec((BM, NCLS), lambda i: (i, 0)),
        out_shape=jax.ShapeDtypeStruct((B, NCLS), jnp.float32),
    )(pooled, W1, b1.reshape(1, HID), W2, b2.reshape(1, NCLS))
    return z


# speedup vs baseline: 23.6963x; 1.4587x over previous
import functools

import jax
import jax.numpy as jnp
from jax import lax
from jax.experimental import pallas as pl
from jax.experimental.pallas import tpu as pltpu
from jax.experimental.pallas import tpu_sc as plsc

B = 16384
L = 200
VOCAB = 1000000
HID = 32
NCP = 16
BMT = 8192
NBLK = 123
VCAP = NBLK * BMT
QROWS = BMT // 8
NC = 2
NS = 16
NW = NC * NS
BPW = B // NW
NB = 8
GROUPS = BPW // NB
IDXG = NB * L
CHUNKS = [(o, 128) for o in range(0, 1536, 128)] + [(1536, 64)]
UNROLL = 10


def _table_kernel(embT_ref, w1_ref, b1_ref, w2p_ref, b2p_ref, o_ref):
    wc = jnp.dot(w1_ref[...], w2p_ref[...],
                 preferred_element_type=jnp.float32)
    bias = jnp.dot(b1_ref[...], w2p_ref[...],
                   preferred_element_type=jnp.float32) + b2p_ref[...]
    res = lax.dot_general(
        embT_ref[...], wc, (((0,), (0,)), ((), ())),
        preferred_element_type=jnp.float32) + bias
    for q in range(8):
        o_ref[:, q * NCP:(q + 1) * NCP] = res[q * QROWS:(q + 1) * QROWS, :]


def _pool_kernel(text_hbm, tbl_hbm, out_hbm,
                 ibuf0, ibuf1, rbuf0, rbuf1, out_v,
                 isem0, isem1, gsem0, gsem1):
    wid = lax.axis_index("s") * NC + lax.axis_index("c")
    inv_l = jnp.float32(1.0 / L)
    zero = jnp.zeros((16,), jnp.float32)
    ibufs = (ibuf0, ibuf1)
    rbufs = (rbuf0, rbuf1)
    isems = (isem0, isem1)
    gsems = (gsem0, gsem1)

    def start_idx(g, b):
        base = (wid * BPW + g * NB) * L
        pltpu.async_copy(text_hbm.at[pl.ds(base, IDXG)], ibufs[b], isems[b])

    def fire_gathers(b):
        pltpu.make_async_copy(text_hbm.at[pl.ds(0, IDXG)],
                              ibufs[b], isems[b]).wait()
        ib = ibufs[b]
        for s in range(IDXG // 16):
            v = ib[pl.ds(s * 16, 16)]
            g = ((v & -8192) | ((v & 1023) << 3) | ((v >> 10) & 7))
            ib[pl.ds(s * 16, 16)] = g
        for off, n in CHUNKS:
            pltpu.async_copy(tbl_hbm.at[ib.at[pl.ds(off, n)]],
                             rbufs[b].at[pl.ds(off, n)], gsems[b])

    def drain_reduce(g, b):
        for off, n in CHUNKS:
            pltpu.make_async_copy(tbl_hbm.at[ibufs[b].at[pl.ds(off, n)]],
                                  rbufs[b].at[pl.ds(off, n)], gsems[b]).wait()
        rbuf = rbufs[b]
        for i in range(NB):
            @pl.loop(0, L, init_carry=zero, unroll=UNROLL)
            def _acc(j, a):
                return a + rbuf[i * L + j]

            out_v[g * NB + i, :] = _acc * inv_l

    start_idx(0, 0)
    start_idx(1, 1)
    fire_gathers(0)

    @pl.loop(0, GROUPS, step=2)
    def _pair(g):
        fire_gathers(1)
        drain_reduce(g, 0)

        @pl.when(g + 2 < GROUPS)
        def _():
            start_idx(g + 2, 0)

        @pl.when(g + 2 < GROUPS)
        def _():
            fire_gathers(0)

        drain_reduce(g + 1, 1)

        @pl.when(g + 3 < GROUPS)
        def _():
            start_idx(g + 3, 1)

    pltpu.sync_copy(out_v, out_hbm.at[pl.ds(wid * BPW, BPW)])


def kernel(text, text_lengths, emb, W1, b1, W2, b2):
    del text_lengths
    text_flat = text.astype(jnp.int32).reshape(B * L)
    embT = emb.T
    NCLS = b2.shape[0]
    W2p = jnp.pad(W2, ((0, 0), (0, NCP - NCLS)))
    b2p = jnp.pad(b2, (0, NCP - NCLS))

    tablec = pl.pallas_call(
        _table_kernel,
        grid=(NBLK,),
        in_specs=[
            pl.BlockSpec((HID, BMT), lambda i: (0, i)),
            pl.BlockSpec((HID, HID), lambda i: (0, 0)),
            pl.BlockSpec((1, HID), lambda i: (0, 0)),
            pl.BlockSpec((HID, NCP), lambda i: (0, 0)),
            pl.BlockSpec((1, NCP), lambda i: (0, 0)),
        ],
        out_specs=pl.BlockSpec((QROWS, 128), lambda i: (i, 0)),
        out_shape=jax.ShapeDtypeStruct((VCAP // 8, 128), jnp.float32),
        compiler_params=pltpu.CompilerParams(fuse_transposed_lhs_in_matmul=True),
    )(embT, W1, b1.reshape(1, HID), W2p, b2p.reshape(1, NCP))
    table2 = tablec.reshape(VCAP, NCP)

    mesh = plsc.VectorSubcoreMesh(core_axis_name="c", subcore_axis_name="s")
    pooled = pl.kernel(
        _pool_kernel,
        out_type=jax.ShapeDtypeStruct((B, NCP), jnp.float32),
        mesh=mesh,
        compiler_params=pltpu.CompilerParams(use_tc_tiling_on_sc=False),
        scratch_types=[
            pltpu.VMEM((IDXG,), jnp.int32),
            pltpu.VMEM((IDXG,), jnp.int32),
            pltpu.VMEM((IDXG, NCP), jnp.float32),
            pltpu.VMEM((IDXG, NCP), jnp.float32),
            pltpu.VMEM((BPW, NCP), jnp.float32),
            pltpu.SemaphoreType.DMA,
            pltpu.SemaphoreType.DMA,
            pltpu.SemaphoreType.DMA,
            pltpu.SemaphoreType.DMA,
        ],
    )(text_flat, table2)

    return pooled[:, :NCLS]
